# baseline (speedup 1.0000x reference)
import functools

import jax
import jax.numpy as jnp
from jax import lax
from jax.experimental import pallas as pl
from jax.experimental.pallas import tpu as pltpu
from jax.experimental.pallas import tpu_sc as plsc

BATCH = 16384
D = 3200
NUM_GROUPS = 10
SLICES_PER_GROUP = 5
SLICE_W = 32
GROUP_W = SLICES_PER_GROUP * SLICE_W

_INFO = plsc.get_sparse_core_info()
_NUM_WORKERS = _INFO.num_cores * _INFO.num_subcores
_ROWS_PER_WORKER = BATCH // _NUM_WORKERS

_MESH = plsc.VectorSubcoreMesh(core_axis_name="c", subcore_axis_name="s")


@functools.partial(
    pl.kernel,
    mesh=_MESH,
    out_type=tuple(
        jax.ShapeDtypeStruct((BATCH, GROUP_W), jnp.float32)
        for _ in range(NUM_GROUPS)
    ),
    scratch_types=[pltpu.SemaphoreType.DMA],
    compiler_params=pltpu.CompilerParams(use_tc_tiling_on_sc=False),
)
def _slice_cat(in_hbm, *out_and_scratch):
    out_hbms = out_and_scratch[:NUM_GROUPS]
    sem = out_and_scratch[NUM_GROUPS]
    wid = lax.axis_index("s") * _INFO.num_cores + lax.axis_index("c")
    base = wid * _ROWS_PER_WORKER
    rows = pl.ds(base, _ROWS_PER_WORKER)
    copies = []
    for g in range(NUM_GROUPS):
        for j in range(SLICES_PER_GROUP):
            chunk = j * NUM_GROUPS + g
            copies.append(
                pltpu.async_copy(
                    in_hbm.at[rows, pl.ds(chunk * SLICE_W, SLICE_W)],
                    out_hbms[g].at[rows, pl.ds(j * SLICE_W, SLICE_W)],
                    sem,
                )
            )
    for c in copies:
        c.wait()


def kernel(input_tensor):
    return _slice_cat(input_tensor)


# speedup vs baseline: 5.1742x; 5.1742x over previous
import functools

import jax
import jax.numpy as jnp
from jax import lax
from jax.experimental import pallas as pl
from jax.experimental.pallas import tpu as pltpu
from jax.experimental.pallas import tpu_sc as plsc

BATCH = 16384
D = 3200
NUM_GROUPS = 10
SLICES_PER_GROUP = 5
SLICE_W = 32
GROUP_W = SLICES_PER_GROUP * SLICE_W
USED_COLS = NUM_GROUPS * GROUP_W
LANES = 16

_INFO = plsc.get_sparse_core_info()
_NUM_WORKERS = _INFO.num_cores * _INFO.num_subcores
_ROWS_PER_WORKER = BATCH // _NUM_WORKERS

_R = 32
_NCHUNK = _ROWS_PER_WORKER // _R

_MESH = plsc.VectorSubcoreMesh(core_axis_name="c", subcore_axis_name="s")


@functools.partial(
    pl.kernel,
    mesh=_MESH,
    out_type=tuple(
        jax.ShapeDtypeStruct((BATCH, GROUP_W), jnp.float32)
        for _ in range(NUM_GROUPS)
    ),
    scratch_types=[
        pltpu.VMEM((_R, USED_COLS), jnp.float32),
        pltpu.VMEM((NUM_GROUPS, _R, GROUP_W), jnp.float32),
        pltpu.SemaphoreType.DMA,
        pltpu.SemaphoreType.DMA,
    ],
    compiler_params=pltpu.CompilerParams(use_tc_tiling_on_sc=False),
)
def _slice_cat(in_hbm, *rest):
    out_hbms = rest[:NUM_GROUPS]
    in_buf, out_buf, sem_in, sem_out = rest[NUM_GROUPS:]
    wid = lax.axis_index("s") * _INFO.num_cores + lax.axis_index("c")
    base = wid * _ROWS_PER_WORKER

    def chunk_body(k, carry):
        row0 = base + k * _R
        pltpu.async_copy(
            in_hbm.at[pl.ds(row0, _R), pl.ds(0, USED_COLS)], in_buf, sem_in
        ).wait()

        def row_body(r, c2):
            for c in range(NUM_GROUPS * SLICES_PER_GROUP):
                g = c % NUM_GROUPS
                j = c // NUM_GROUPS
                for h in range(SLICE_W // LANES):
                    v = in_buf[r, pl.ds(c * SLICE_W + h * LANES, LANES)]
                    out_buf[g, r, pl.ds(j * SLICE_W + h * LANES, LANES)] = v
            return c2

        lax.fori_loop(0, _R, row_body, 0)

        copies = [
            pltpu.async_copy(
                out_buf.at[g], out_hbms[g].at[pl.ds(row0, _R), :], sem_out
            )
            for g in range(NUM_GROUPS)
        ]
        for c in copies:
            c.wait()
        return carry

    lax.fori_loop(0, _NCHUNK, chunk_body, 0)


def kernel(input_tensor):
    return _slice_cat(input_tensor)


# speedup vs baseline: 5.7232x; 1.1061x over previous
import functools

import jax
import jax.numpy as jnp
from jax import lax
from jax.experimental import pallas as pl
from jax.experimental.pallas import tpu as pltpu
from jax.experimental.pallas import tpu_sc as plsc

BATCH = 16384
D = 3200
NUM_GROUPS = 10
SLICES_PER_GROUP = 5
SLICE_W = 32
GROUP_W = SLICES_PER_GROUP * SLICE_W
USED_COLS = NUM_GROUPS * GROUP_W
LANES = 16

_INFO = plsc.get_sparse_core_info()
_NUM_WORKERS = _INFO.num_cores * _INFO.num_subcores
_ROWS_PER_WORKER = BATCH // _NUM_WORKERS

_R = 16
_NCHUNK = _ROWS_PER_WORKER // _R
_RUNROLL = 2

_MESH = plsc.VectorSubcoreMesh(core_axis_name="c", subcore_axis_name="s")


@functools.partial(
    pl.kernel,
    mesh=_MESH,
    out_type=tuple(
        jax.ShapeDtypeStruct((BATCH, GROUP_W), jnp.float32)
        for _ in range(NUM_GROUPS)
    ),
    scratch_types=[
        pltpu.VMEM((2, _R, USED_COLS), jnp.float32),
        pltpu.VMEM((2, NUM_GROUPS, _R, GROUP_W), jnp.float32),
        pltpu.SemaphoreType.DMA,
        pltpu.SemaphoreType.DMA,
        pltpu.SemaphoreType.DMA,
        pltpu.SemaphoreType.DMA,
    ],
    compiler_params=pltpu.CompilerParams(use_tc_tiling_on_sc=False),
)
def _slice_cat(in_hbm, *rest):
    out_hbms = rest[:NUM_GROUPS]
    in_buf, out_buf = rest[NUM_GROUPS:NUM_GROUPS + 2]
    sems_in = rest[NUM_GROUPS + 2:NUM_GROUPS + 4]
    sems_out = rest[NUM_GROUPS + 4:NUM_GROUPS + 6]
    wid = lax.axis_index("s") * _INFO.num_cores + lax.axis_index("c")
    base = wid * _ROWS_PER_WORKER

    def in_copy(k, b):
        row0 = base + k * _R
        return pltpu.make_async_copy(
            in_hbm.at[pl.ds(row0, _R), pl.ds(0, USED_COLS)],
            in_buf.at[b],
            sems_in[b],
        )

    def out_copies(k, b):
        row0 = base + k * _R
        return [
            pltpu.make_async_copy(
                out_buf.at[b, g],
                out_hbms[g].at[pl.ds(row0, _R), :],
                sems_out[b],
            )
            for g in range(NUM_GROUPS)
        ]

    def shuffle(b):
        def row_body(r2, c2):
            for dr in range(_RUNROLL):
                r = r2 * _RUNROLL + dr
                for c in range(NUM_GROUPS * SLICES_PER_GROUP):
                    g = c % NUM_GROUPS
                    j = c // NUM_GROUPS
                    for h in range(SLICE_W // LANES):
                        v = in_buf[b, r, pl.ds(c * SLICE_W + h * LANES, LANES)]
                        out_buf[b, g, r, pl.ds(j * SLICE_W + h * LANES, LANES)] = v
            return c2

        lax.fori_loop(0, _R // _RUNROLL, row_body, 0)

    def step(k, b):
        @pl.when(k + 1 < _NCHUNK)
        def _():
            in_copy(k + 1, 1 - b).start()

        in_copy(k, b).wait()

        @pl.when(k >= 2)
        def _():
            for c in out_copies(k - 2, b):
                c.wait()

        shuffle(b)
        for c in out_copies(k, b):
            c.start()

    in_copy(0, 0).start()

    def pair_body(k0, carry):
        step(2 * k0, 0)
        step(2 * k0 + 1, 1)
        return carry

    lax.fori_loop(0, _NCHUNK // 2, pair_body, 0)

    for b in (0, 1):
        for c in out_copies(_NCHUNK - 2 + b, b):
            c.wait()


def kernel(input_tensor):
    return _slice_cat(input_tensor)


# speedup vs baseline: 6.9113x; 1.2076x over previous
import functools

import jax
import jax.numpy as jnp
from jax import lax
from jax.experimental import pallas as pl
from jax.experimental.pallas import tpu as pltpu
from jax.experimental.pallas import tpu_sc as plsc

BATCH = 16384
D = 3200
NUM_GROUPS = 10
SLICES_PER_GROUP = 5
SLICE_W = 32
GROUP_W = SLICES_PER_GROUP * SLICE_W
USED_COLS = NUM_GROUPS * GROUP_W
LANES = 16

_INFO = plsc.get_sparse_core_info()
_NUM_WORKERS = _INFO.num_cores * _INFO.num_subcores
_ROWS_PER_WORKER = BATCH // _NUM_WORKERS

_R = 16
_NCHUNK = _ROWS_PER_WORKER // _R
_RUNROLL = 2

_MESH = plsc.VectorSubcoreMesh(core_axis_name="c", subcore_axis_name="s")


@functools.partial(
    pl.kernel,
    mesh=_MESH,
    out_type=tuple(
        jax.ShapeDtypeStruct((BATCH, GROUP_W), jnp.float32)
        for _ in range(NUM_GROUPS)
    ),
    scratch_types=[
        pltpu.VMEM((2, _R, USED_COLS), jnp.float32),
        pltpu.VMEM((2, NUM_GROUPS, _R, GROUP_W), jnp.float32),
        pltpu.SemaphoreType.DMA,
        pltpu.SemaphoreType.DMA,
        pltpu.SemaphoreType.DMA,
        pltpu.SemaphoreType.DMA,
    ],
    compiler_params=pltpu.CompilerParams(use_tc_tiling_on_sc=False),
)
def _slice_cat(in_hbm, *rest):
    out_hbms = rest[:NUM_GROUPS]
    in_buf, out_buf = rest[NUM_GROUPS:NUM_GROUPS + 2]
    sems_in = rest[NUM_GROUPS + 2:NUM_GROUPS + 4]
    sems_out = rest[NUM_GROUPS + 4:NUM_GROUPS + 6]
    wid = lax.axis_index("s") * _INFO.num_cores + lax.axis_index("c")
    base = wid * _ROWS_PER_WORKER

    def in_copy(k, b):
        row0 = base + k * _R
        return pltpu.make_async_copy(
            in_hbm.at[pl.ds(row0, _R), pl.ds(0, USED_COLS)],
            in_buf.at[b],
            sems_in[b],
        )

    def out_copies(k, b):
        row0 = base + k * _R
        return [
            pltpu.make_async_copy(
                out_buf.at[b, g],
                out_hbms[g].at[pl.ds(row0, _R), :],
                sems_out[b],
            )
            for g in range(NUM_GROUPS)
        ]

    def shuffle(b):
        @plsc.parallel_loop(0, _R, unroll=_RUNROLL)
        def _(r):
            for c in range(NUM_GROUPS * SLICES_PER_GROUP):
                g = c % NUM_GROUPS
                j = c // NUM_GROUPS
                for h in range(SLICE_W // LANES):
                    v = in_buf[b, r, pl.ds(c * SLICE_W + h * LANES, LANES)]
                    out_buf[b, g, r, pl.ds(j * SLICE_W + h * LANES, LANES)] = v

    def step(k, b):
        @pl.when(k + 1 < _NCHUNK)
        def _():
            in_copy(k + 1, 1 - b).start()

        in_copy(k, b).wait()

        @pl.when(k >= 2)
        def _():
            for c in out_copies(k - 2, b):
                c.wait()

        shuffle(b)
        for c in out_copies(k, b):
            c.start()

    in_copy(0, 0).start()

    def pair_body(k0, carry):
        step(2 * k0, 0)
        step(2 * k0 + 1, 1)
        return carry

    lax.fori_loop(0, _NCHUNK // 2, pair_body, 0)

    for b in (0, 1):
        for c in out_copies(_NCHUNK - 2 + b, b):
            c.wait()


def kernel(input_tensor):
    return _slice_cat(input_tensor)
